# baseline (speedup 1.0000x reference)
import functools

import jax
import jax.numpy as jnp
from jax import lax
from jax.experimental import pallas as pl
from jax.experimental.pallas import tpu as pltpu
from jax.experimental.pallas import tpu_sc as plsc

_BOX = 10.0
_K = 16

_SC_CORES = 2
_SC_SUBCORES = 16
_SC_WORKERS = _SC_CORES * _SC_SUBCORES
_CHUNK = 128


def _topk_body(npt_ref, pp_ref, idx_ref, r_ref, dx_ref, dy_ref, dz_ref):
    nx = npt_ref[0]
    ny = npt_ref[1]
    nz = npt_ref[2]
    px = pp_ref[:, 0:1]
    py = pp_ref[:, 1:2]
    pz = pp_ref[:, 2:3]

    def wrap(d):
        return d - _BOX * jnp.round(d / _BOX)

    dx = wrap(px - nx)
    dy = wrap(py - ny)
    dz = wrap(pz - nz)
    r = jnp.sqrt(dx * dx + dy * dy + dz * dz)

    m = r.shape[1]
    lane = lax.broadcasted_iota(jnp.int32, r.shape, 1).astype(jnp.float32)
    icols, rcols, xcols, ycols, zcols = [], [], [], [], []
    for _ in range(_K):
        mn = jnp.min(r, axis=1, keepdims=True)
        cand = jnp.where(r == mn, lane, float(m))
        mi = jnp.min(cand, axis=1, keepdims=True)
        onehot = lane == mi
        icols.append(mi)
        rcols.append(mn)
        xcols.append(jnp.sum(jnp.where(onehot, dx, 0.0), axis=1, keepdims=True))
        ycols.append(jnp.sum(jnp.where(onehot, dy, 0.0), axis=1, keepdims=True))
        zcols.append(jnp.sum(jnp.where(onehot, dz, 0.0), axis=1, keepdims=True))
        r = jnp.where(onehot, jnp.inf, r)
    m_idx = jnp.concatenate(icols, axis=1)

    bb = pp_ref.shape[0]
    row = lax.broadcasted_iota(jnp.int32, (bb, _K), 0)
    gb = pl.program_id(0) * bb + row
    idx_ref[...] = gb * jnp.int32(m) + m_idx.astype(jnp.int32)
    r_ref[...] = jnp.concatenate(rcols, axis=1)
    dx_ref[...] = jnp.concatenate(xcols, axis=1)
    dy_ref[...] = jnp.concatenate(ycols, axis=1)
    dz_ref[...] = jnp.concatenate(zcols, axis=1)


def _topk(np_t, particle_pos, block_b=256, interpret=False):
    b = particle_pos.shape[0]
    m = np_t.shape[2]
    kspec = pl.BlockSpec((block_b, _K), lambda i: (i, 0))
    return pl.pallas_call(
        _topk_body,
        grid=(b // block_b,),
        in_specs=[
            pl.BlockSpec((3, block_b, m), lambda i: (0, i, 0)),
            pl.BlockSpec((block_b, 3), lambda i: (i, 0)),
        ],
        out_specs=[kspec] * 5,
        out_shape=[
            jax.ShapeDtypeStruct((b, _K), jnp.int32),
            jax.ShapeDtypeStruct((b, _K), jnp.float32),
            jax.ShapeDtypeStruct((b, _K), jnp.float32),
            jax.ShapeDtypeStruct((b, _K), jnp.float32),
            jax.ShapeDtypeStruct((b, _K), jnp.float32),
        ],
        interpret=interpret,
    )(np_t, particle_pos)


def _sc_gather(table8, idx3):
    n_chunks = idx3.shape[1]
    per_w = n_chunks * _CHUNK
    n = idx3.shape[0] * per_w
    mesh = plsc.VectorSubcoreMesh(core_axis_name="c", subcore_axis_name="s")

    @functools.partial(
        pl.kernel,
        mesh=mesh,
        compiler_params=pltpu.CompilerParams(use_tc_tiling_on_sc=False),
        out_type=jax.ShapeDtypeStruct((n, 8), jnp.float32),
        scratch_types=[
            pltpu.VMEM((n_chunks, _CHUNK), jnp.int32),
            pltpu.VMEM((_CHUNK, 8), jnp.float32),
            pltpu.SemaphoreType.DMA,
        ],
    )
    def gather_kernel(t_hbm, idx_hbm, out_hbm, idx_v, rows, sem):
        wid = lax.axis_index("s") * _SC_CORES + lax.axis_index("c")
        base = wid * per_w
        pltpu.sync_copy(idx_hbm.at[wid], idx_v)

        def body(g, carry):
            pltpu.async_copy(t_hbm.at[idx_v.at[g]], rows, sem).wait()
            pltpu.sync_copy(rows, out_hbm.at[pl.ds(base + g * _CHUNK, _CHUNK)])
            return carry

        lax.fori_loop(0, n_chunks, body, 0)

    return gather_kernel(table8, idx3)


def _feat_mlp_body(j_ref, g_ref, pr_ref, r_in_ref, dx_in_ref, dy_in_ref,
                   dz_in_ref, w1_ref, b1_ref, w2_ref, b2_ref, w3_ref, b3_ref,
                   w4_ref, b4_ref, w5_ref, b5_ref, out_ref):
    n = j_ref.shape[0]

    off = j_ref[...] & jnp.int32(7)
    nine = jnp.zeros((n, 9), jnp.float32)
    for s in range(8):
        nine = jnp.where(off == s, g_ref[:, s:s + 9], nine)

    r = r_in_ref[...]
    inv_r = 1.0 / r
    d = [dx_in_ref[...], dy_in_ref[...], dz_in_ref[...]]
    u = [d[c] / r for c in range(3)]

    p = [[pr_ref[:, 3 * i + c:3 * i + c + 1] for c in range(3)] for i in range(3)]
    q = [[nine[:, 3 * i + c:3 * i + c + 1] for c in range(3)] for i in range(3)]

    prod = [[[p[i][c] * q[l][c] for c in range(3)] for l in range(3)]
            for i in range(3)]
    dot = [[prod[i][l][0] + prod[i][l][1] + prod[i][l][2]
            for l in range(3)] for i in range(3)]
    elem_norm = [[jnp.sqrt(prod[i][l][0] * prod[i][l][0]
                           + prod[i][l][1] * prod[i][l][1]
                           + prod[i][l][2] * prod[i][l][2])
                  for l in range(3)] for i in range(3)]
    cross = [[p[i][1] * q[i][2] - p[i][2] * q[i][1],
              p[i][2] * q[i][0] - p[i][0] * q[i][2],
              p[i][0] * q[i][1] - p[i][1] * q[i][0]] for i in range(3)]
    cross_norm = [jnp.sqrt(cross[i][0] * cross[i][0]
                           + cross[i][1] * cross[i][1]
                           + cross[i][2] * cross[i][2]) for i in range(3)]
    rel = [[p[0][i] * q[0][l] + p[1][i] * q[1][l] + p[2][i] * q[2][l]
            for l in range(3)] for i in range(3)]
    rbf_p = [u[0] * p[j][0] + u[1] * p[j][1] + u[2] * p[j][2] for j in range(3)]
    rbf_n = [u[0] * q[j][0] + u[1] * q[j][1] + u[2] * q[j][2] for j in range(3)]
    tr = rel[0][0] + rel[1][1] + rel[2][2]
    ca = jnp.clip((tr - 1.0) / 2.0, -1.0 + 1e-6, 1.0 - 1e-6)
    ang = 2.0 * jnp.arctan2(jnp.sqrt(1.0 - ca), jnp.sqrt(1.0 + ca))

    cols = [r, inv_r] + u
    cols += [dot[i][l] for i in range(3) for l in range(3)]
    cols += [prod[i][l][c] for i in range(3) for l in range(3) for c in range(3)]
    cols += [elem_norm[i][l] for i in range(3) for l in range(3)]
    cols += [cross[i][c] for i in range(3) for c in range(3)]
    cols += cross_norm
    cols += [rel[i][l] for i in range(3) for l in range(3)]
    cols += rbf_p + rbf_n + [ang]
    feats = jnp.concatenate(cols, axis=1)

    def pe_tanh(x, w, b):
        x3 = x.reshape(n // _K, _K, x.shape[1])
        xm = jnp.max(x3, axis=1, keepdims=True)
        xc = (x3 - xm).reshape(n, x.shape[1])
        return jnp.tanh(
            jnp.dot(xc, w[...], preferred_element_type=jnp.float32) + b[...])

    h = pe_tanh(feats, w1_ref, b1_ref)
    h = pe_tanh(h, w2_ref, b2_ref)
    h = pe_tanh(h, w3_ref, b3_ref)
    pooled = jnp.max(h.reshape(n // _K, _K, h.shape[1]), axis=1)
    o = jnp.tanh(
        jnp.dot(pooled, w4_ref[...], preferred_element_type=jnp.float32)
        + b4_ref[...])
    out_ref[...] = (
        jnp.dot(o, w5_ref[...], preferred_element_type=jnp.float32)
        + b5_ref[...])


def _feat_mlp(j_col, g16, pr_rep, r_col, dx_col, dy_col, dz_col, weights,
              block_b=64, interpret=False):
    w1, b1, w2, b2, w3, b3, w4, b4, w5, b5 = weights
    bk = j_col.shape[0]
    b = bk // _K
    nrow = block_b * _K

    def rows(i):
        return (i, 0)

    def whole(i):
        return (0, 0)

    return pl.pallas_call(
        _feat_mlp_body,
        grid=(b // block_b,),
        in_specs=[
            pl.BlockSpec((nrow, 1), rows),
            pl.BlockSpec((nrow, 16), rows),
            pl.BlockSpec((nrow, 9), rows),
            pl.BlockSpec((nrow, 1), rows),
            pl.BlockSpec((nrow, 1), rows),
            pl.BlockSpec((nrow, 1), rows),
            pl.BlockSpec((nrow, 1), rows),
            pl.BlockSpec(w1.shape, whole),
            pl.BlockSpec(b1.shape, whole),
            pl.BlockSpec(w2.shape, whole),
            pl.BlockSpec(b2.shape, whole),
            pl.BlockSpec(w3.shape, whole),
            pl.BlockSpec(b3.shape, whole),
            pl.BlockSpec(w4.shape, whole),
            pl.BlockSpec(b4.shape, whole),
            pl.BlockSpec(w5.shape, whole),
            pl.BlockSpec(b5.shape, whole),
        ],
        out_specs=pl.BlockSpec((block_b, 1), rows),
        out_shape=jax.ShapeDtypeStruct((b, 1), jnp.float32),
        interpret=interpret,
    )(j_col, g16, pr_rep, r_col, dx_col, dy_col, dz_col,
      w1, b1, w2, b2, w3, b3, w4, b4, w5, b5)


def kernel(particle_pos, neighbors_pos, particle_R, neighbors_R,
           W1, b1, W2, b2, W3, b3, W4, b4, W5, b5):
    b, m, _ = neighbors_pos.shape
    bk = b * _K

    np_t = jnp.transpose(neighbors_pos, (2, 0, 1))
    j_idx, r_sel, dx_sel, dy_sel, dz_sel = _topk(np_t, particle_pos)

    a = (j_idx.reshape(-1) * jnp.int32(9)) >> 3
    idx2 = jnp.stack([a, a + jnp.int32(1)], axis=-1)
    n_chunks = (2 * bk) // (_SC_WORKERS * _CHUNK)
    idx3 = idx2.reshape(_SC_WORKERS, n_chunks, _CHUNK)
    table8 = neighbors_R.reshape(b * m * 9 // 8, 8)
    g8 = _sc_gather(table8, idx3)
    g16 = g8.reshape(bk, 16)

    pr_rep = jnp.broadcast_to(
        particle_R.reshape(b, 1, 9), (b, _K, 9)).reshape(bk, 9)

    weights = (W1, b1.reshape(1, -1), W2, b2.reshape(1, -1),
               W3, b3.reshape(1, -1), W4, b4.reshape(1, -1),
               W5, b5.reshape(1, -1))
    return _feat_mlp(j_idx.reshape(bk, 1), g16, pr_rep,
                     r_sel.reshape(bk, 1), dx_sel.reshape(bk, 1),
                     dy_sel.reshape(bk, 1), dz_sel.reshape(bk, 1), weights)


# speedup vs baseline: 1.0050x; 1.0050x over previous
import functools

import jax
import jax.numpy as jnp
from jax import lax
from jax.experimental import pallas as pl
from jax.experimental.pallas import tpu as pltpu
from jax.experimental.pallas import tpu_sc as plsc

_BOX = 10.0
_K = 16

_SC_CORES = 2
_SC_SUBCORES = 16
_SC_WORKERS = _SC_CORES * _SC_SUBCORES
_CHUNK = 128


def _topk_body(npf_ref, pp_ref, idx_ref, r_ref, dx_ref, dy_ref, dz_ref):
    mm = npf_ref.shape[1]
    m = mm // 3
    x3 = npf_ref[...].reshape(npf_ref.shape[0], m, 3)
    nx = x3[:, :, 0]
    ny = x3[:, :, 1]
    nz = x3[:, :, 2]
    px = pp_ref[:, 0:1]
    py = pp_ref[:, 1:2]
    pz = pp_ref[:, 2:3]

    def wrap(d):
        return d - _BOX * jnp.round(d / _BOX)

    dx = wrap(px - nx)
    dy = wrap(py - ny)
    dz = wrap(pz - nz)
    r = jnp.sqrt(dx * dx + dy * dy + dz * dz)

    lane = lax.broadcasted_iota(jnp.int32, r.shape, 1).astype(jnp.float32)
    icols, rcols, xcols, ycols, zcols = [], [], [], [], []
    for _ in range(_K):
        mn = jnp.min(r, axis=1, keepdims=True)
        cand = jnp.where(r == mn, lane, float(m))
        mi = jnp.min(cand, axis=1, keepdims=True)
        onehot = lane == mi
        icols.append(mi)
        rcols.append(mn)
        xcols.append(jnp.sum(jnp.where(onehot, dx, 0.0), axis=1, keepdims=True))
        ycols.append(jnp.sum(jnp.where(onehot, dy, 0.0), axis=1, keepdims=True))
        zcols.append(jnp.sum(jnp.where(onehot, dz, 0.0), axis=1, keepdims=True))
        r = jnp.where(onehot, jnp.inf, r)
    m_idx = jnp.concatenate(icols, axis=1)

    bb = pp_ref.shape[0]
    row = lax.broadcasted_iota(jnp.int32, (bb, _K), 0)
    gb = pl.program_id(0) * bb + row
    idx_ref[...] = gb * jnp.int32(m) + m_idx.astype(jnp.int32)
    r_ref[...] = jnp.concatenate(rcols, axis=1)
    dx_ref[...] = jnp.concatenate(xcols, axis=1)
    dy_ref[...] = jnp.concatenate(ycols, axis=1)
    dz_ref[...] = jnp.concatenate(zcols, axis=1)


def _topk(np_flat, particle_pos, block_b=64, interpret=False):
    b = particle_pos.shape[0]
    mm = np_flat.shape[1]
    kspec = pl.BlockSpec((block_b, _K), lambda i: (i, 0))
    return pl.pallas_call(
        _topk_body,
        grid=(b // block_b,),
        in_specs=[
            pl.BlockSpec((block_b, mm), lambda i: (i, 0)),
            pl.BlockSpec((block_b, 3), lambda i: (i, 0)),
        ],
        out_specs=[kspec] * 5,
        out_shape=[
            jax.ShapeDtypeStruct((b, _K), jnp.int32),
            jax.ShapeDtypeStruct((b, _K), jnp.float32),
            jax.ShapeDtypeStruct((b, _K), jnp.float32),
            jax.ShapeDtypeStruct((b, _K), jnp.float32),
            jax.ShapeDtypeStruct((b, _K), jnp.float32),
        ],
        interpret=interpret,
    )(np_flat, particle_pos)


def _sc_gather(table8, idx3):
    n_chunks = idx3.shape[1]
    per_w = n_chunks * _CHUNK
    n = idx3.shape[0] * per_w
    mesh = plsc.VectorSubcoreMesh(core_axis_name="c", subcore_axis_name="s")

    @functools.partial(
        pl.kernel,
        mesh=mesh,
        compiler_params=pltpu.CompilerParams(use_tc_tiling_on_sc=False),
        out_type=jax.ShapeDtypeStruct((n, 8), jnp.float32),
        scratch_types=[
            pltpu.VMEM((n_chunks, _CHUNK), jnp.int32),
            pltpu.VMEM((_CHUNK, 8), jnp.float32),
            pltpu.SemaphoreType.DMA,
        ],
    )
    def gather_kernel(t_hbm, idx_hbm, out_hbm, idx_v, rows, sem):
        wid = lax.axis_index("s") * _SC_CORES + lax.axis_index("c")
        base = wid * per_w
        pltpu.sync_copy(idx_hbm.at[wid], idx_v)

        def body(g, carry):
            pltpu.async_copy(t_hbm.at[idx_v.at[g]], rows, sem).wait()
            pltpu.sync_copy(rows, out_hbm.at[pl.ds(base + g * _CHUNK, _CHUNK)])
            return carry

        lax.fori_loop(0, n_chunks, body, 0)

    return gather_kernel(table8, idx3)


def _feat_mlp_body(j_ref, g_ref, pr_ref, r_in_ref, dx_in_ref, dy_in_ref,
                   dz_in_ref, w1_ref, b1_ref, w2_ref, b2_ref, w3_ref, b3_ref,
                   w4_ref, b4_ref, w5_ref, b5_ref, out_ref):
    n = j_ref.shape[0]

    off = j_ref[...] & jnp.int32(7)
    nine = jnp.zeros((n, 9), jnp.float32)
    for s in range(8):
        nine = jnp.where(off == s, g_ref[:, s:s + 9], nine)

    nb = n // _K
    pr = jnp.broadcast_to(
        pr_ref[...][:, None, :], (nb, _K, 9)).reshape(n, 9)

    r = r_in_ref[...]
    inv_r = 1.0 / r
    d = [dx_in_ref[...], dy_in_ref[...], dz_in_ref[...]]
    u = [d[c] / r for c in range(3)]

    p = [[pr[:, 3 * i + c:3 * i + c + 1] for c in range(3)] for i in range(3)]
    q = [[nine[:, 3 * i + c:3 * i + c + 1] for c in range(3)] for i in range(3)]

    prod = [[[p[i][c] * q[l][c] for c in range(3)] for l in range(3)]
            for i in range(3)]
    dot = [[prod[i][l][0] + prod[i][l][1] + prod[i][l][2]
            for l in range(3)] for i in range(3)]
    elem_norm = [[jnp.sqrt(prod[i][l][0] * prod[i][l][0]
                           + prod[i][l][1] * prod[i][l][1]
                           + prod[i][l][2] * prod[i][l][2])
                  for l in range(3)] for i in range(3)]
    cross = [[p[i][1] * q[i][2] - p[i][2] * q[i][1],
              p[i][2] * q[i][0] - p[i][0] * q[i][2],
              p[i][0] * q[i][1] - p[i][1] * q[i][0]] for i in range(3)]
    cross_norm = [jnp.sqrt(cross[i][0] * cross[i][0]
                           + cross[i][1] * cross[i][1]
                           + cross[i][2] * cross[i][2]) for i in range(3)]
    rel = [[p[0][i] * q[0][l] + p[1][i] * q[1][l] + p[2][i] * q[2][l]
            for l in range(3)] for i in range(3)]
    rbf_p = [u[0] * p[j][0] + u[1] * p[j][1] + u[2] * p[j][2] for j in range(3)]
    rbf_n = [u[0] * q[j][0] + u[1] * q[j][1] + u[2] * q[j][2] for j in range(3)]
    tr = rel[0][0] + rel[1][1] + rel[2][2]
    ca = jnp.clip((tr - 1.0) / 2.0, -1.0 + 1e-6, 1.0 - 1e-6)
    ang = 2.0 * jnp.arctan2(jnp.sqrt(1.0 - ca), jnp.sqrt(1.0 + ca))

    cols = [r, inv_r] + u
    cols += [dot[i][l] for i in range(3) for l in range(3)]
    cols += [prod[i][l][c] for i in range(3) for l in range(3) for c in range(3)]
    cols += [elem_norm[i][l] for i in range(3) for l in range(3)]
    cols += [cross[i][c] for i in range(3) for c in range(3)]
    cols += cross_norm
    cols += [rel[i][l] for i in range(3) for l in range(3)]
    cols += rbf_p + rbf_n + [ang]
    feats = jnp.concatenate(cols, axis=1)

    def pe_tanh(x, w, b):
        x3 = x.reshape(n // _K, _K, x.shape[1])
        xm = jnp.max(x3, axis=1, keepdims=True)
        xc = (x3 - xm).reshape(n, x.shape[1])
        return jnp.tanh(
            jnp.dot(xc, w[...], preferred_element_type=jnp.float32) + b[...])

    h = pe_tanh(feats, w1_ref, b1_ref)
    h = pe_tanh(h, w2_ref, b2_ref)
    h = pe_tanh(h, w3_ref, b3_ref)
    pooled = jnp.max(h.reshape(n // _K, _K, h.shape[1]), axis=1)
    o = jnp.tanh(
        jnp.dot(pooled, w4_ref[...], preferred_element_type=jnp.float32)
        + b4_ref[...])
    out_ref[...] = (
        jnp.dot(o, w5_ref[...], preferred_element_type=jnp.float32)
        + b5_ref[...])


def _feat_mlp(j_col, g16, pr_mat, r_col, dx_col, dy_col, dz_col, weights,
              block_b=64, interpret=False):
    w1, b1, w2, b2, w3, b3, w4, b4, w5, b5 = weights
    bk = j_col.shape[0]
    b = bk // _K
    nrow = block_b * _K

    def rows(i):
        return (i, 0)

    def whole(i):
        return (0, 0)

    return pl.pallas_call(
        _feat_mlp_body,
        grid=(b // block_b,),
        in_specs=[
            pl.BlockSpec((nrow, 1), rows),
            pl.BlockSpec((nrow, 16), rows),
            pl.BlockSpec((block_b, 9), rows),
            pl.BlockSpec((nrow, 1), rows),
            pl.BlockSpec((nrow, 1), rows),
            pl.BlockSpec((nrow, 1), rows),
            pl.BlockSpec((nrow, 1), rows),
            pl.BlockSpec(w1.shape, whole),
            pl.BlockSpec(b1.shape, whole),
            pl.BlockSpec(w2.shape, whole),
            pl.BlockSpec(b2.shape, whole),
            pl.BlockSpec(w3.shape, whole),
            pl.BlockSpec(b3.shape, whole),
            pl.BlockSpec(w4.shape, whole),
            pl.BlockSpec(b4.shape, whole),
            pl.BlockSpec(w5.shape, whole),
            pl.BlockSpec(b5.shape, whole),
        ],
        out_specs=pl.BlockSpec((block_b, 1), rows),
        out_shape=jax.ShapeDtypeStruct((b, 1), jnp.float32),
        interpret=interpret,
    )(j_col, g16, pr_mat, r_col, dx_col, dy_col, dz_col,
      w1, b1, w2, b2, w3, b3, w4, b4, w5, b5)


def kernel(particle_pos, neighbors_pos, particle_R, neighbors_R,
           W1, b1, W2, b2, W3, b3, W4, b4, W5, b5):
    b, m, _ = neighbors_pos.shape
    bk = b * _K

    np_flat = neighbors_pos.reshape(b, m * 3)
    j_idx, r_sel, dx_sel, dy_sel, dz_sel = _topk(np_flat, particle_pos)

    a = (j_idx.reshape(-1) * jnp.int32(9)) >> 3
    idx2 = jnp.stack([a, a + jnp.int32(1)], axis=-1)
    n_chunks = (2 * bk) // (_SC_WORKERS * _CHUNK)
    idx3 = idx2.reshape(_SC_WORKERS, n_chunks, _CHUNK)
    table8 = neighbors_R.reshape(b * m * 9 // 8, 8)
    g8 = _sc_gather(table8, idx3)
    g16 = g8.reshape(bk, 16)

    weights = (W1, b1.reshape(1, -1), W2, b2.reshape(1, -1),
               W3, b3.reshape(1, -1), W4, b4.reshape(1, -1),
               W5, b5.reshape(1, -1))
    return _feat_mlp(j_idx.reshape(bk, 1), g16, particle_R.reshape(b, 9),
                     r_sel.reshape(bk, 1), dx_sel.reshape(bk, 1),
                     dy_sel.reshape(bk, 1), dz_sel.reshape(bk, 1), weights)
